# baseline (device time: 38052 ns/iter reference)
import jax
import jax.numpy as jnp
from jax import lax
from jax.experimental import pallas as pl
from jax.experimental.pallas import tpu as pltpu

N_DEV = 16
LOG2_N = 4
BLK = 32


def kernel(x, Wq, Wo, K_ext, V_ext):
    B, Sq, D = x.shape
    _, Skv, Hkv, Dh = K_ext.shape
    Dq = Wq.shape[1]
    Hq = Dq // Dh
    G = Hq // Hkv
    Do = Wo.shape[1]
    R = B * Sq
    W = Dq + Hq
    OFF = [0, 256, 384, 448]

    def body(x_ref, wq_ref, wo_ref, k_ref, v_ref, out_ref,
             o_acc, o_tx, o_rx, out_bf, o_ssem, o_rsem):
        me = lax.axis_index("i")
        pos = (((me & 1) << 3) | ((me & 2) << 1)
               | ((me & 4) >> 1) | ((me & 8) >> 3))
        even = (me & 1) == 0
        pending = []

        def compute_partial(b, lo, hi):
            n = hi - lo
            xb = x_ref[b][lo:hi, :].astype(jnp.bfloat16)
            q = jnp.dot(xb, wq_ref[...].astype(jnp.bfloat16),
                        preferred_element_type=jnp.float32) * 0.125
            kb = k_ref[b].reshape(Skv, Hkv * Dh).astype(jnp.bfloat16)
            vb = v_ref[b].reshape(Skv, Hkv * Dh).astype(jnp.bfloat16)
            for g in range(Hkv):
                qg = jnp.concatenate(
                    [q[:, (g * G + i) * Dh:(g * G + i + 1) * Dh]
                     for i in range(G)], axis=0).astype(jnp.bfloat16)
                kh = kb[:, g * Dh:(g + 1) * Dh]
                vh = vb[:, g * Dh:(g + 1) * Dh]
                s = lax.dot_general(qg, kh, (((1,), (1,)), ((), ())),
                                    preferred_element_type=jnp.float32)
                p_ = jnp.exp(s)
                lsum = jnp.sum(p_, axis=1, keepdims=True)
                og = jnp.dot(p_.astype(jnp.bfloat16), vh,
                             preferred_element_type=jnp.float32)
                for i in range(G):
                    h = g * G + i
                    o_acc[b * Sq + lo:b * Sq + hi, Dq + h:Dq + h + 1] = lsum[
                        i * n:(i + 1) * n, :]
                    o_acc[b * Sq + lo:b * Sq + hi, h * Dh:(h + 1) * Dh] = og[
                        i * n:(i + 1) * n, :]

        def rs_round(k):
            p = jnp.bitwise_xor(me, 1 << k)
            nblk = 8 >> k
            rows = nblk * BLK
            s_keep = (pos >> (3 - k)) << (3 - k)
            s_send = jnp.bitwise_xor(s_keep, nblk)
            o_tx[OFF[k]:OFF[k] + rows, :] = o_acc[
                pl.ds(s_send * BLK, rows), :].astype(jnp.bfloat16)
            o_rdma = pltpu.make_async_remote_copy(
                src_ref=o_tx.at[pl.ds(OFF[k], rows)],
                dst_ref=o_rx.at[pl.ds(OFF[k], rows)],
                send_sem=o_ssem.at[k], recv_sem=o_rsem.at[k],
                device_id=(p,), device_id_type=pl.DeviceIdType.MESH)
            o_rdma.start()
            pending.append(o_rdma)
            return o_rdma, rows, s_keep, OFF[k]

        def rs_finish(o_rdma, rows, s_keep, off):
            o_rdma.wait_recv()
            o_acc[pl.ds(s_keep * BLK, rows), :] = (
                o_acc[pl.ds(s_keep * BLK, rows), :]
                + o_rx[pl.ds(off, rows), :].astype(jnp.float32))

        bit1z = (me & 2) == 0
        nev = jnp.logical_not(even)
        nb1 = jnp.logical_not(bit1z)
        H = Sq // 2
        QF, QS = (H, Sq), (0, H)

        bs = jnp.where(even, 1, 0)
        rel_f = jnp.where(bit1z, H, 0)
        rel_s = H - rel_f
        s_keep1 = (pos >> 2) << 2
        s_send1 = jnp.bitwise_xor(s_keep1, 4)
        p0 = jnp.bitwise_xor(me, 1)
        p1p = jnp.bitwise_xor(me, 2)
        RX2 = 480

        def quarter(cond, b, qr):
            @pl.when(cond)
            def _():
                compute_partial(b, *qr)

        def send(src, dst, idx, partner):
            rdma = pltpu.make_async_remote_copy(
                src_ref=src, dst_ref=dst,
                send_sem=o_ssem.at[idx], recv_sem=o_rsem.at[idx],
                device_id=(partner,), device_id_type=pl.DeviceIdType.MESH)
            rdma.start()
            pending.append(rdma)
            return rdma

        quarter(jnp.logical_and(even, bit1z), 1, QF)
        quarter(jnp.logical_and(even, nb1), 1, QS)
        quarter(jnp.logical_and(nev, bit1z), 0, QF)
        quarter(jnp.logical_and(nev, nb1), 0, QS)

        bar = pltpu.get_barrier_semaphore()
        for r in range(LOG2_N):
            p = jnp.bitwise_xor(me, 1 << r)
            pl.semaphore_signal(bar, inc=1, device_id=(p,),
                                device_id_type=pl.DeviceIdType.MESH)
        pl.semaphore_wait(bar, LOG2_N)

        o_tx[0:H, :] = o_acc[pl.ds(bs * Sq + rel_f, H), :].astype(jnp.bfloat16)
        r0a = send(o_tx.at[pl.ds(0, H)], o_rx.at[pl.ds(rel_f, H)], 0, p0)

        quarter(jnp.logical_and(even, bit1z), 1, QS)
        quarter(jnp.logical_and(even, nb1), 1, QF)
        quarter(jnp.logical_and(nev, bit1z), 0, QS)
        quarter(jnp.logical_and(nev, nb1), 0, QF)

        o_tx[H:2 * H, :] = o_acc[pl.ds(bs * Sq + rel_s, H), :].astype(
            jnp.bfloat16)
        r0b = send(o_tx.at[pl.ds(H, H)], o_rx.at[pl.ds(rel_s, H)],
                   3 * LOG2_N + 1, p0)

        quarter(jnp.logical_and(even, bit1z), 0, QF)
        quarter(jnp.logical_and(even, nb1), 0, QS)
        quarter(jnp.logical_and(nev, bit1z), 1, QF)
        quarter(jnp.logical_and(nev, nb1), 1, QS)

        o_tx[OFF[1]:OFF[1] + H, :] = o_acc[pl.ds(s_send1 * BLK, H), :].astype(
            jnp.bfloat16)
        r1p1 = send(o_tx.at[pl.ds(OFF[1], H)], o_rx.at[pl.ds(OFF[1], H)],
                    1, p1p)
        r0a.wait_recv()
        r1p2 = send(o_rx.at[pl.ds(rel_f, H)], o_rx.at[pl.ds(RX2, H)],
                    3 * LOG2_N, p1p)

        quarter(jnp.logical_and(even, bit1z), 0, QS)
        quarter(jnp.logical_and(even, nb1), 0, QF)
        quarter(jnp.logical_and(nev, bit1z), 1, QS)
        quarter(jnp.logical_and(nev, nb1), 1, QF)

        r0b.wait_recv()
        o_acc[pl.ds(s_keep1 * BLK, H), :] = (
            o_acc[pl.ds(s_keep1 * BLK, H), :]
            + o_rx[pl.ds(rel_s, H), :].astype(jnp.float32))
        r1p1.wait_recv()
        r1p2.wait_recv()
        o_acc[pl.ds(s_keep1 * BLK, H), :] = (
            o_acc[pl.ds(s_keep1 * BLK, H), :]
            + o_rx[pl.ds(OFF[1], H), :].astype(jnp.float32)
            + o_rx[pl.ds(RX2, H), :].astype(jnp.float32))

        for k in range(2, LOG2_N):
            rs_finish(*rs_round(k))

        o_red = o_acc[pl.ds(pos * BLK, BLK), :]
        blocks = []
        for h in range(Hq):
            lcol = o_red[:, Dq + h:Dq + h + 1]
            blocks.append(o_red[:, h * Dh:(h + 1) * Dh] / lcol)
        onorm = jnp.concatenate(blocks, axis=1).astype(jnp.bfloat16)
        myout = jnp.dot(onorm, wo_ref[...].astype(jnp.bfloat16),
                        preferred_element_type=jnp.float32)
        out_bf[pl.ds(pos * BLK, BLK), :] = myout.astype(jnp.bfloat16)

        def ag_send(region_start, nblk, partner, sem_idx):
            rdma = pltpu.make_async_remote_copy(
                src_ref=out_bf.at[pl.ds(region_start * BLK, nblk * BLK)],
                dst_ref=out_bf.at[pl.ds(region_start * BLK, nblk * BLK)],
                send_sem=o_ssem.at[sem_idx], recv_sem=o_rsem.at[sem_idx],
                device_id=(partner,), device_id_type=pl.DeviceIdType.MESH)
            rdma.start()
            pending.append(rdma)
            return rdma

        def win(k):
            return (pos >> k) << k

        partners = [jnp.bitwise_xor(me, 1 << (3 - k)) for k in range(LOG2_N)]
        r0 = ag_send(win(0), 1, partners[0], LOG2_N)
        p1 = {1: ag_send(win(0), 1, partners[1], LOG2_N + 1)}
        p2 = {}
        r0.wait_recv()
        p2[1] = ag_send(jnp.bitwise_xor(win(0), 1), 1, partners[1],
                        2 * LOG2_N + 1)
        p1[2] = ag_send(win(1), 2, partners[2], LOG2_N + 2)
        p1[1].wait_recv()
        p2[1].wait_recv()
        p2[2] = ag_send(jnp.bitwise_xor(win(1), 2), 2, partners[2],
                        2 * LOG2_N + 2)
        p1[3] = ag_send(win(2), 4, partners[3], LOG2_N + 3)
        p1[2].wait_recv()
        p2[2].wait_recv()
        p2[3] = ag_send(jnp.bitwise_xor(win(2), 4), 4, partners[3],
                        2 * LOG2_N + 3)
        p1[3].wait_recv()
        p2[3].wait_recv()

        out_ref[...] = out_bf[...].astype(jnp.float32)

        for dsc in pending:
            dsc.wait_send()

    out_flat = pl.pallas_call(
        body,
        out_shape=jax.ShapeDtypeStruct((R, Do), jnp.float32),
        in_specs=[pl.BlockSpec(memory_space=pltpu.VMEM)] * 5,
        out_specs=pl.BlockSpec(memory_space=pltpu.VMEM),
        scratch_shapes=[
            pltpu.VMEM((R, W), jnp.float32),
            pltpu.VMEM((480, W), jnp.bfloat16),
            pltpu.VMEM((608, W), jnp.bfloat16),
            pltpu.VMEM((R, Do), jnp.bfloat16),
            pltpu.SemaphoreType.DMA((3 * LOG2_N + 2,)),
            pltpu.SemaphoreType.DMA((3 * LOG2_N + 2,)),
        ],
        compiler_params=pltpu.CompilerParams(collective_id=0),
    )(x, Wq, Wo, K_ext, V_ext)
    return out_flat.reshape(B, Sq, Do)
